# baseline (speedup 1.0000x reference)
import functools

import jax
import jax.numpy as jnp
from jax import lax
from jax.experimental import pallas as pl
from jax.experimental.pallas import tpu as pltpu
from jax.experimental.pallas import tpu_sc as plsc

DIM = 1024
N_TOKENS = 10
BATCH = 4
SEQ = 2058
N_GATHER = BATCH * (SEQ - N_TOKENS)

NC = 1
NS = 16
NW = NC * NS
ROWS_PER_W = N_GATHER // NW
CHUNK = 32
N_CHUNKS = ROWS_PER_W // CHUNK
W_PER_BATCH = NW // BATCH


@functools.partial(
    pl.kernel,
    mesh=plsc.VectorSubcoreMesh(
        core_axis_name="c", subcore_axis_name="s", num_cores=NC),
    out_type=jax.ShapeDtypeStruct((BATCH, SEQ, DIM), jnp.float32),
    scratch_types=[
        pltpu.VMEM((ROWS_PER_W,), jnp.int32),
        pltpu.VMEM((N_CHUNKS, CHUNK), jnp.int32),
        pltpu.VMEM((CHUNK, DIM), jnp.float32),
        pltpu.VMEM((CHUNK, DIM), jnp.float32),
        pltpu.VMEM((CHUNK, DIM), jnp.float32),
        pltpu.VMEM((16, DIM), jnp.float32),
        pltpu.VMEM((1, 16), jnp.int32),
        pltpu.SemaphoreType.DMA,
        pltpu.SemaphoreType.DMA,
        pltpu.SemaphoreType.DMA,
        pltpu.SemaphoreType.DMA,
        pltpu.SemaphoreType.DMA,
        pltpu.SemaphoreType.DMA,
        pltpu.SemaphoreType.DMA,
    ],
)
def _soft_embed(table_hbm, idx_hbm, learned_hbm, oidx_hbm, lidx_hbm, out_hbm,
                idx_v, oidx_v, buf0, buf1, buf2, lbuf, lidx_v,
                gsem0, gsem1, gsem2, ssem0, ssem1, ssem2, lsem):
    wid = lax.axis_index("s") * NC + lax.axis_index("c")
    gbase = wid * ROWS_PER_W
    b = wid // W_PER_BATCH

    @pl.when(wid % W_PER_BATCH == 0)
    def _():
        pltpu.sync_copy(learned_hbm, lbuf)
        pltpu.sync_copy(lidx_hbm.at[b], lidx_v)
        pltpu.async_copy(lbuf, out_hbm.at[b].at[lidx_v.at[0]], lsem).wait()

    pltpu.sync_copy(idx_hbm.at[pl.ds(gbase, ROWS_PER_W)], idx_v)
    pltpu.sync_copy(oidx_hbm.at[wid], oidx_v)

    NBUF = 3
    bufs = (buf0, buf1, buf2)
    gsems = (gsem0, gsem1, gsem2)
    ssems = (ssem0, ssem1, ssem2)

    def start_gather(j):
        return pltpu.async_copy(
            table_hbm.at[idx_v.at[pl.ds(j * CHUNK, CHUNK)]],
            bufs[j % NBUF], gsems[j % NBUF])

    def start_scatter(j):
        return pltpu.async_copy(
            bufs[j % NBUF], out_hbm.at[b].at[oidx_v.at[j]], ssems[j % NBUF])

    gcopies = [None] * N_CHUNKS
    scopies = [None] * N_CHUNKS
    for j in range(min(NBUF, N_CHUNKS)):
        gcopies[j] = start_gather(j)
    for j in range(N_CHUNKS):
        gcopies[j].wait()
        scopies[j] = start_scatter(j)
        nxt = j + NBUF - 1
        if j >= 1 and nxt < N_CHUNKS:
            scopies[j - 1].wait()
            gcopies[nxt] = start_gather(nxt)
    for j in range(max(0, N_CHUNKS - NBUF), N_CHUNKS):
        scopies[j].wait()


def _make_row_indices():
    w = jnp.arange(NW, dtype=jnp.int32)
    obase = N_TOKENS + (w % W_PER_BATCH) * ROWS_PER_W
    oidx = (obase[:, None, None]
            + (jnp.arange(N_CHUNKS, dtype=jnp.int32) * CHUNK)[None, :, None]
            + jnp.arange(CHUNK, dtype=jnp.int32)[None, None, :])
    lrow = jnp.arange(16, dtype=jnp.int32)
    lidx = jnp.broadcast_to(lrow[None, None, :], (BATCH, 1, 16))
    return oidx, lidx


def kernel(tokens, wte_weight, learned_embedding):
    idx = tokens[:, N_TOKENS:].reshape(-1).astype(jnp.int32)
    oidx, lidx = _make_row_indices()
    learned_pad = jnp.pad(learned_embedding, ((0, 16 - N_TOKENS), (0, 0)))
    return _soft_embed(wte_weight, idx, learned_pad, oidx, lidx)


# speedup vs baseline: 1.5227x; 1.5227x over previous
import functools

import jax
import jax.numpy as jnp
from jax import lax
from jax.experimental import pallas as pl
from jax.experimental.pallas import tpu as pltpu
from jax.experimental.pallas import tpu_sc as plsc

DIM = 1024
N_TOKENS = 10
BATCH = 4
SEQ = 2058
N_GATHER = BATCH * (SEQ - N_TOKENS)

NC = 2
NS = 16
NW = NC * NS
ROWS_PER_W = N_GATHER // NW
CHUNK = 32
N_CHUNKS = ROWS_PER_W // CHUNK
W_PER_BATCH = NW // BATCH


@functools.partial(
    pl.kernel,
    mesh=plsc.VectorSubcoreMesh(
        core_axis_name="c", subcore_axis_name="s", num_cores=NC),
    out_type=jax.ShapeDtypeStruct((BATCH, 2064, DIM), jnp.float32),
    scratch_types=[
        pltpu.VMEM((ROWS_PER_W,), jnp.int32),
        pltpu.VMEM((N_CHUNKS, CHUNK), jnp.int32),
        pltpu.VMEM((CHUNK, DIM), jnp.float32),
        pltpu.VMEM((CHUNK, DIM), jnp.float32),
        pltpu.VMEM((CHUNK, DIM), jnp.float32),
        pltpu.VMEM((16, DIM), jnp.float32),
        pltpu.VMEM((1, 16), jnp.int32),
        pltpu.SemaphoreType.DMA,
        pltpu.SemaphoreType.DMA,
        pltpu.SemaphoreType.DMA,
        pltpu.SemaphoreType.DMA,
        pltpu.SemaphoreType.DMA,
        pltpu.SemaphoreType.DMA,
        pltpu.SemaphoreType.DMA,
    ],
)
def _soft_embed(table_hbm, idx_hbm, learned_hbm, oidx_hbm, lidx_hbm, out_hbm,
                idx_v, oidx_v, buf0, buf1, buf2, lbuf, lidx_v,
                gsem0, gsem1, gsem2, ssem0, ssem1, ssem2, lsem):
    wid = lax.axis_index("s") * NC + lax.axis_index("c")
    gbase = wid * ROWS_PER_W
    b = wid // W_PER_BATCH

    @pl.when(wid % W_PER_BATCH == 0)
    def _():
        pltpu.sync_copy(learned_hbm, lbuf)
        pltpu.sync_copy(lidx_hbm.at[b], lidx_v)
        pltpu.async_copy(lbuf, out_hbm.at[b].at[lidx_v.at[0]], lsem).wait()

    pltpu.sync_copy(idx_hbm.at[pl.ds(gbase, ROWS_PER_W)], idx_v)
    pltpu.sync_copy(oidx_hbm.at[wid], oidx_v)

    NBUF = 3
    bufs = (buf0, buf1, buf2)
    gsems = (gsem0, gsem1, gsem2)
    ssems = (ssem0, ssem1, ssem2)

    def start_gather(j):
        return pltpu.async_copy(
            table_hbm.at[idx_v.at[pl.ds(j * CHUNK, CHUNK)]],
            bufs[j % NBUF], gsems[j % NBUF])

    def start_scatter(j):
        return pltpu.async_copy(
            bufs[j % NBUF], out_hbm.at[b].at[oidx_v.at[j]], ssems[j % NBUF])

    gcopies = [None] * N_CHUNKS
    scopies = [None] * N_CHUNKS
    for j in range(min(NBUF, N_CHUNKS)):
        gcopies[j] = start_gather(j)
    for j in range(N_CHUNKS):
        gcopies[j].wait()
        scopies[j] = start_scatter(j)
        nxt = j + NBUF - 1
        if j >= 1 and nxt < N_CHUNKS:
            scopies[j - 1].wait()
            gcopies[nxt] = start_gather(nxt)
    for j in range(max(0, N_CHUNKS - NBUF), N_CHUNKS):
        scopies[j].wait()


def _make_row_indices():
    w = jnp.arange(NW, dtype=jnp.int32)
    obase = N_TOKENS + (w % W_PER_BATCH) * ROWS_PER_W
    oidx = (obase[:, None, None]
            + (jnp.arange(N_CHUNKS, dtype=jnp.int32) * CHUNK)[None, :, None]
            + jnp.arange(CHUNK, dtype=jnp.int32)[None, None, :])
    lrow = jnp.arange(16, dtype=jnp.int32)
    lidx = jnp.broadcast_to(lrow[None, None, :], (BATCH, 1, 16))
    return oidx, lidx


def kernel(tokens, wte_weight, learned_embedding):
    idx = tokens[:, N_TOKENS:].reshape(-1).astype(jnp.int32)
    oidx, lidx = _make_row_indices()
    learned_pad = jnp.pad(learned_embedding, ((0, 16 - N_TOKENS), (0, 0)))
    out = _soft_embed(wte_weight, idx, learned_pad, oidx, lidx)
    return out[:, :SEQ, :]
